# baseline (speedup 1.0000x reference)
import functools

import jax
import jax.numpy as jnp
from jax import lax
from jax.experimental import pallas as pl
from jax.experimental.pallas import tpu as pltpu
from jax.experimental.pallas import tpu_sc as plsc

DIM = 32
K = 8192
N = 8192
BM = 256
WK = 2048

_INFO = plsc.get_sparse_core_info()
_NC, _NS, _L = _INFO.num_cores, _INFO.num_subcores, _INFO.num_lanes
NW = _NC * _NS
BPW = N // NW


def _argmin_body(x_ref, cb_ref, x2_ref, c2_ref, idx_ref, acc_ref, run_ref):
    w = pl.program_id(1)
    x = x_ref[...]
    cb = cb_ref[...]
    x2 = x2_ref[...][:, None]
    c2 = c2_ref[...][None, :]

    s = lax.dot_general(x, cb, (((1,), (1,)), ((), ())),
                        preferred_element_type=jnp.float32)
    d2 = x2 + c2 - 2.0 * s
    dist = jnp.sqrt(jnp.maximum(d2, 0.0))

    m_w = jnp.min(dist, axis=1, keepdims=True)
    iota_w = lax.broadcasted_iota(jnp.int32, dist.shape, 1) + w * WK
    cand_w = jnp.where(dist == m_w, iota_w, K)
    idx_w = jnp.min(cand_w, axis=1, keepdims=True)

    @pl.when(w == 0)
    def _init():
        acc_ref[...] = jnp.full((BM, 1), jnp.inf, jnp.float32)
        run_ref[...] = jnp.zeros((BM, 1), jnp.int32)

    acc = acc_ref[...]
    idx = run_ref[...]
    take = (m_w < acc) | ((m_w == acc) & (idx_w < idx))
    acc_ref[...] = jnp.where(take, m_w, acc).astype(jnp.bfloat16).astype(
        jnp.float32)
    new_idx = jnp.where(take, idx_w, idx)
    run_ref[...] = new_idx

    @pl.when(w == pl.num_programs(1) - 1)
    def _fin():
        idx_ref[...] = new_idx[:, 0]


@jax.jit
def _tc_argmin(x_flat, codebook, x2, c2):
    return pl.pallas_call(
        _argmin_body,
        grid=(N // BM, K // WK),
        in_specs=[
            pl.BlockSpec((BM, DIM), lambda i, w: (i, 0)),
            pl.BlockSpec((WK, DIM), lambda i, w: (w, 0)),
            pl.BlockSpec((BM,), lambda i, w: (i,)),
            pl.BlockSpec((WK,), lambda i, w: (w,)),
        ],
        out_specs=pl.BlockSpec((BM,), lambda i, w: (i,)),
        out_shape=jax.ShapeDtypeStruct((N,), jnp.int32),
        scratch_shapes=[
            pltpu.VMEM((BM, 1), jnp.float32),
            pltpu.VMEM((BM, 1), jnp.int32),
        ],
    )(x_flat, codebook, x2, c2)


def _sc_body(idx_hbm, x_hbm, cb_hbm, q_out, part_out, idx_v, rows_v, x_v,
             q_v, part_v, sem):
    wid = lax.axis_index("s") * _NC + lax.axis_index("c")
    base = wid * BPW
    pltpu.sync_copy(idx_hbm.at[pl.ds(base, BPW)], idx_v)
    pltpu.async_copy(cb_hbm.at[idx_v], rows_v, sem).wait()
    pltpu.sync_copy(x_hbm.at[pl.ds(base, BPW)], x_v)

    def row(i, acc):
        for j in range(DIM // _L):
            sl = pl.ds(j * _L, _L)
            r = rows_v[i, sl]
            xv = x_v[i, sl]
            dv = r - xv
            q_v[i, sl] = xv + dv
            acc = acc + dv * dv
        return acc

    acc = lax.fori_loop(0, BPW, row, jnp.zeros((_L,), jnp.float32))
    part_v[...] = acc
    pltpu.sync_copy(q_v, q_out.at[pl.ds(base, BPW)])
    pltpu.sync_copy(part_v, part_out.at[wid])


@jax.jit
def _sc_gather(idx, x_flat, cb):
    mesh = plsc.VectorSubcoreMesh(core_axis_name="c", subcore_axis_name="s")
    f = functools.partial(
        pl.kernel,
        out_type=[
            jax.ShapeDtypeStruct((N, DIM), jnp.float32),
            jax.ShapeDtypeStruct((NW, _L), jnp.float32),
        ],
        mesh=mesh,
        scratch_types=[
            pltpu.VMEM((BPW,), jnp.int32),
            pltpu.VMEM((BPW, 128), jnp.float32),
            pltpu.VMEM((BPW, DIM), jnp.float32),
            pltpu.VMEM((BPW, DIM), jnp.float32),
            pltpu.VMEM((_L,), jnp.float32),
            pltpu.SemaphoreType.DMA,
        ],
    )(_sc_body)
    return f(idx, x_flat, cb)


def kernel(x, codebook):
    b, dim, seq = x.shape
    xp = jnp.transpose(x, (0, 2, 1))
    x_flat = xp.reshape(-1, dim)
    x2 = jnp.sum(x_flat * x_flat, axis=1)
    c2 = jnp.sum(codebook * codebook, axis=1)
    idx = _tc_argmin(x_flat, codebook, x2, c2)
    cb_pad = jnp.pad(codebook, ((0, 0), (0, 128 - dim)))
    q_flat, parts = _sc_gather(idx, x_flat, cb_pad)
    quantized = q_flat.reshape(b, seq, dim).transpose(0, 2, 1)
    loss = jnp.sum(parts) * (1.0 / (N * DIM))
    return quantized, idx.reshape(b, seq), loss


# speedup vs baseline: 1.1907x; 1.1907x over previous
import functools

import jax
import jax.numpy as jnp
from jax import lax
from jax.experimental import pallas as pl
from jax.experimental.pallas import tpu as pltpu
from jax.experimental.pallas import tpu_sc as plsc

DIM = 32
K = 8192
N = 8192
BM = 512
WK = 2048

_INFO = plsc.get_sparse_core_info()
_NC, _NS, _L = _INFO.num_cores, _INFO.num_subcores, _INFO.num_lanes
NW = _NC * _NS
BPW = N // NW


def _argmin_body(x_ref, cb_ref, x2_ref, c2_ref, idx_ref):
    x = x_ref[...]
    cb = cb_ref[...]
    x2 = x2_ref[...][:, None]
    c2 = c2_ref[...][None, :]

    s = lax.dot_general(x, cb, (((1,), (1,)), ((), ())),
                        preferred_element_type=jnp.float32)
    d2 = x2 + c2 - 2.0 * s
    dist = jnp.sqrt(jnp.maximum(d2, 0.0))

    acc = jnp.full((BM, 1), jnp.inf, jnp.float32)
    idx = jnp.zeros((BM, 1), jnp.int32)
    for w in range(K // WK):
        blk = dist[:, w * WK:(w + 1) * WK]
        m_w = jnp.min(blk, axis=1, keepdims=True)
        iota_w = lax.broadcasted_iota(jnp.int32, blk.shape, 1) + w * WK
        cand_w = jnp.where(blk == m_w, iota_w, K)
        idx_w = jnp.min(cand_w, axis=1, keepdims=True)
        take = (m_w < acc) | ((m_w == acc) & (idx_w < idx))
        acc = jnp.where(take, m_w, acc).astype(jnp.bfloat16).astype(
            jnp.float32)
        idx = jnp.where(take, idx_w, idx)

    idx_ref[...] = idx[:, 0]


@jax.jit
def _tc_argmin(x_flat, codebook, x2, c2):
    return pl.pallas_call(
        _argmin_body,
        grid=(N // BM,),
        in_specs=[
            pl.BlockSpec((BM, DIM), lambda i: (i, 0)),
            pl.BlockSpec((K, DIM), lambda i: (0, 0)),
            pl.BlockSpec((BM,), lambda i: (i,)),
            pl.BlockSpec((K,), lambda i: (0,)),
        ],
        out_specs=pl.BlockSpec((BM,), lambda i: (i,)),
        out_shape=jax.ShapeDtypeStruct((N,), jnp.int32),
    )(x_flat, codebook, x2, c2)


def _sc_body(idx_hbm, x_hbm, cb_hbm, q_out, part_out, idx_v, rows_v, x_v,
             q_v, part_v, sem):
    wid = lax.axis_index("s") * _NC + lax.axis_index("c")
    base = wid * BPW
    pltpu.sync_copy(idx_hbm.at[pl.ds(base, BPW)], idx_v)
    pltpu.async_copy(cb_hbm.at[idx_v], rows_v, sem).wait()
    pltpu.sync_copy(x_hbm.at[pl.ds(base, BPW)], x_v)

    def row(i, acc):
        for j in range(DIM // _L):
            sl = pl.ds(j * _L, _L)
            r = rows_v[i, sl]
            xv = x_v[i, sl]
            dv = r - xv
            q_v[i, sl] = xv + dv
            acc = acc + dv * dv
        return acc

    acc = lax.fori_loop(0, BPW, row, jnp.zeros((_L,), jnp.float32))
    part_v[...] = acc
    pltpu.sync_copy(q_v, q_out.at[pl.ds(base, BPW)])
    pltpu.sync_copy(part_v, part_out.at[wid])


@jax.jit
def _sc_gather(idx, x_flat, cb):
    mesh = plsc.VectorSubcoreMesh(core_axis_name="c", subcore_axis_name="s")
    f = functools.partial(
        pl.kernel,
        out_type=[
            jax.ShapeDtypeStruct((N, DIM), jnp.float32),
            jax.ShapeDtypeStruct((NW, _L), jnp.float32),
        ],
        mesh=mesh,
        scratch_types=[
            pltpu.VMEM((BPW,), jnp.int32),
            pltpu.VMEM((BPW, 128), jnp.float32),
            pltpu.VMEM((BPW, DIM), jnp.float32),
            pltpu.VMEM((BPW, DIM), jnp.float32),
            pltpu.VMEM((_L,), jnp.float32),
            pltpu.SemaphoreType.DMA,
        ],
    )(_sc_body)
    return f(idx, x_flat, cb)


def kernel(x, codebook):
    b, dim, seq = x.shape
    xp = jnp.transpose(x, (0, 2, 1))
    x_flat = xp.reshape(-1, dim)
    x2 = jnp.sum(x_flat * x_flat, axis=1)
    c2 = jnp.sum(codebook * codebook, axis=1)
    idx = _tc_argmin(x_flat, codebook, x2, c2)
    cb_pad = jnp.pad(codebook, ((0, 0), (0, 128 - dim)))
    q_flat, parts = _sc_gather(idx, x_flat, cb_pad)
    quantized = q_flat.reshape(b, seq, dim).transpose(0, 2, 1)
    loss = jnp.sum(parts) * (1.0 / (N * DIM))
    return quantized, idx.reshape(b, seq), loss


# speedup vs baseline: 1.2373x; 1.0391x over previous
import functools

import jax
import jax.numpy as jnp
from jax import lax
from jax.experimental import pallas as pl
from jax.experimental.pallas import tpu as pltpu
from jax.experimental.pallas import tpu_sc as plsc

DIM = 32
K = 8192
N = 8192
BM = 1024
WK = 2048

_INFO = plsc.get_sparse_core_info()
_NC, _NS, _L = _INFO.num_cores, _INFO.num_subcores, _INFO.num_lanes
NW = _NC * _NS
BPW = N // NW


def _argmin_body(x_ref, cb_ref, x2_ref, c2_ref, idx_ref):
    x = x_ref[...]
    cb = cb_ref[...]
    x2 = x2_ref[...][:, None]
    c2 = c2_ref[...][None, :]

    s = lax.dot_general(x, cb, (((1,), (1,)), ((), ())),
                        preferred_element_type=jnp.float32)
    d2 = x2 + c2 - 2.0 * s
    dist = jnp.sqrt(jnp.maximum(d2, 0.0))

    acc = jnp.full((BM, 1), jnp.inf, jnp.float32)
    idx = jnp.zeros((BM, 1), jnp.int32)
    for w in range(K // WK):
        blk = dist[:, w * WK:(w + 1) * WK]
        m_w = jnp.min(blk, axis=1, keepdims=True)
        iota_w = lax.broadcasted_iota(jnp.int32, blk.shape, 1) + w * WK
        cand_w = jnp.where(blk == m_w, iota_w, K)
        idx_w = jnp.min(cand_w, axis=1, keepdims=True)
        take = (m_w < acc) | ((m_w == acc) & (idx_w < idx))
        acc = jnp.where(take, m_w, acc).astype(jnp.bfloat16).astype(
            jnp.float32)
        idx = jnp.where(take, idx_w, idx)

    idx_ref[...] = idx[:, 0]


@jax.jit
def _tc_argmin(x_flat, codebook, x2, c2):
    return pl.pallas_call(
        _argmin_body,
        grid=(N // BM,),
        in_specs=[
            pl.BlockSpec((BM, DIM), lambda i: (i, 0)),
            pl.BlockSpec((K, DIM), lambda i: (0, 0)),
            pl.BlockSpec((BM,), lambda i: (i,)),
            pl.BlockSpec((K,), lambda i: (0,)),
        ],
        out_specs=pl.BlockSpec((BM,), lambda i: (i,)),
        out_shape=jax.ShapeDtypeStruct((N,), jnp.int32),
    )(x_flat, codebook, x2, c2)


def _sc_body(idx_hbm, x_hbm, cb_hbm, q_out, part_out, idx_v, rows_v, x_v,
             q_v, part_v, sem):
    wid = lax.axis_index("s") * _NC + lax.axis_index("c")
    base = wid * BPW
    pltpu.sync_copy(idx_hbm.at[pl.ds(base, BPW)], idx_v)
    pltpu.async_copy(cb_hbm.at[idx_v], rows_v, sem).wait()
    pltpu.sync_copy(x_hbm.at[pl.ds(base, BPW)], x_v)

    def row(i, acc):
        for j in range(DIM // _L):
            sl = pl.ds(j * _L, _L)
            r = rows_v[i, sl]
            xv = x_v[i, sl]
            dv = r - xv
            q_v[i, sl] = xv + dv
            acc = acc + dv * dv
        return acc

    acc = lax.fori_loop(0, BPW, row, jnp.zeros((_L,), jnp.float32))
    part_v[...] = acc
    pltpu.sync_copy(q_v, q_out.at[pl.ds(base, BPW)])
    pltpu.sync_copy(part_v, part_out.at[wid])


@jax.jit
def _sc_gather(idx, x_flat, cb):
    mesh = plsc.VectorSubcoreMesh(core_axis_name="c", subcore_axis_name="s")
    f = functools.partial(
        pl.kernel,
        out_type=[
            jax.ShapeDtypeStruct((N, DIM), jnp.float32),
            jax.ShapeDtypeStruct((NW, _L), jnp.float32),
        ],
        mesh=mesh,
        scratch_types=[
            pltpu.VMEM((BPW,), jnp.int32),
            pltpu.VMEM((BPW, 128), jnp.float32),
            pltpu.VMEM((BPW, DIM), jnp.float32),
            pltpu.VMEM((BPW, DIM), jnp.float32),
            pltpu.VMEM((_L,), jnp.float32),
            pltpu.SemaphoreType.DMA,
        ],
    )(_sc_body)
    return f(idx, x_flat, cb)


def kernel(x, codebook):
    b, dim, seq = x.shape
    xp = jnp.transpose(x, (0, 2, 1))
    x_flat = xp.reshape(-1, dim)
    x2 = jnp.sum(x_flat * x_flat, axis=1)
    c2 = jnp.sum(codebook * codebook, axis=1)
    idx = _tc_argmin(x_flat, codebook, x2, c2)
    cb_pad = jnp.pad(codebook, ((0, 0), (0, 128 - dim)))
    q_flat, parts = _sc_gather(idx, x_flat, cb_pad)
    quantized = q_flat.reshape(b, seq, dim).transpose(0, 2, 1)
    loss = jnp.sum(parts) * (1.0 / (N * DIM))
    return quantized, idx.reshape(b, seq), loss
